# Initial kernel scaffold; baseline (speedup 1.0000x reference)
import jax
import jax.numpy as jnp
from jax.experimental import pallas as pl


def kernel(input_ids, embed_weight, head_weight, head_bias):
    raise NotImplementedError("write your pallas kernel here")


# speedup vs baseline: 1.7922x; 1.7922x over previous
import functools

import jax
import jax.numpy as jnp
from jax import lax
from jax.experimental import pallas as pl
from jax.experimental.pallas import tpu as pltpu
from jax.experimental.pallas import tpu_sc as plsc

V = 32
D = 8
NC = 2
NS = 16
NW = NC * NS


def _m_table_body(e_ref, ht_ref, b_ref, m_ref):
    m_ref[...] = (
        jnp.dot(e_ref[...], ht_ref[...], preferred_element_type=jnp.float32)
        + b_ref[...]
    )


def _compute_m_table(embed_weight, head_weight_t, head_bias_row):
    return pl.pallas_call(
        _m_table_body,
        out_shape=jax.ShapeDtypeStruct((V, V), jnp.float32),
    )(embed_weight, head_weight_t, head_bias_row)


@functools.partial(jax.jit, static_argnames=("b", "chunk"))
def _sc_gather(m, ids, b, chunk):
    b_per_w = b // NW
    n_chunks = b_per_w // chunk
    mesh = plsc.VectorSubcoreMesh(core_axis_name="c", subcore_axis_name="s")

    @functools.partial(
        pl.kernel,
        mesh=mesh,
        out_type=jax.ShapeDtypeStruct((b, V), jnp.float32),
        scratch_types=[
            pltpu.VMEM((chunk,), jnp.int32),
            pltpu.VMEM((chunk, V), jnp.float32),
            pltpu.SemaphoreType.DMA,
        ],
        compiler_params=pltpu.CompilerParams(use_tc_tiling_on_sc=False),
    )
    def gather_kernel(m_hbm, idx_hbm, out_hbm, idx_v, rows_v, sem):
        wid = lax.axis_index("s") * NC + lax.axis_index("c")
        base = wid * b_per_w
        for c in range(n_chunks):
            off = base + c * chunk
            pltpu.sync_copy(idx_hbm.at[pl.ds(off, chunk)], idx_v)
            pltpu.async_copy(m_hbm.at[idx_v], rows_v, sem).wait()
            pltpu.sync_copy(rows_v, out_hbm.at[pl.ds(off, chunk)])

    return gather_kernel(m, ids)


def kernel(input_ids, embed_weight, head_weight, head_bias):
    bt, sl = input_ids.shape
    b = bt * sl
    m = _compute_m_table(
        embed_weight, head_weight.T, head_bias.reshape(1, V)
    )
    ids = input_ids.reshape(b).astype(jnp.int32)
    out = _sc_gather(m, ids, b, 1600)
    return out.reshape(bt, sl, V)


# speedup vs baseline: 4.1999x; 2.3434x over previous
import functools

import jax
import jax.numpy as jnp
from jax import lax
from jax.experimental import pallas as pl
from jax.experimental.pallas import tpu as pltpu
from jax.experimental.pallas import tpu_sc as plsc

V = 32
D = 8
NC = 2
NS = 16
NW = NC * NS
R_REP = 16


def _m_table_body(e_ref, ht_ref, b_ref, m_ref):
    m_ref[...] = (
        jnp.dot(e_ref[...], ht_ref[...], preferred_element_type=jnp.float32)
        + b_ref[...]
    )


def _compute_m_table(embed_weight, head_weight_t, head_bias_row):
    return pl.pallas_call(
        _m_table_body,
        grid=(R_REP,),
        in_specs=[
            pl.BlockSpec((V, D), lambda r: (0, 0)),
            pl.BlockSpec((D, V), lambda r: (0, 0)),
            pl.BlockSpec((1, V), lambda r: (0, 0)),
        ],
        out_specs=pl.BlockSpec((V, V), lambda r: (r, 0)),
        out_shape=jax.ShapeDtypeStruct((R_REP * V, V), jnp.float32),
    )(embed_weight, head_weight_t, head_bias_row)


@functools.partial(jax.jit, static_argnames=("b", "chunk"))
def _sc_gather(m, ids, b, chunk):
    b_per_w = b // NW
    n_chunks = b_per_w // chunk
    mesh = plsc.VectorSubcoreMesh(core_axis_name="c", subcore_axis_name="s")

    @functools.partial(
        pl.kernel,
        mesh=mesh,
        out_type=jax.ShapeDtypeStruct((b, V), jnp.float32),
        scratch_types=[
            pltpu.VMEM((chunk,), jnp.int32),
            pltpu.VMEM((chunk, V), jnp.float32),
            pltpu.SemaphoreType.DMA,
        ],
        compiler_params=pltpu.CompilerParams(use_tc_tiling_on_sc=False),
    )
    def gather_kernel(m_hbm, idx_hbm, out_hbm, idx_v, rows_v, sem):
        wid = lax.axis_index("s") * NC + lax.axis_index("c")
        base = wid * b_per_w
        rep_off = lax.iota(jnp.int32, 16) * V
        for c in range(n_chunks):
            off = base + c * chunk
            pltpu.sync_copy(idx_hbm.at[pl.ds(off, chunk)], idx_v)

            def spread(i, carry):
                sl = pl.ds(i * 16, 16)
                idx_v[sl] = idx_v[sl] + rep_off
                return carry

            lax.fori_loop(0, chunk // 16, spread, 0)
            pltpu.async_copy(m_hbm.at[idx_v], rows_v, sem).wait()
            pltpu.sync_copy(rows_v, out_hbm.at[pl.ds(off, chunk)])

    return gather_kernel(m, ids)


def kernel(input_ids, embed_weight, head_weight, head_bias):
    bt, sl = input_ids.shape
    b = bt * sl
    m = _compute_m_table(
        embed_weight, head_weight.T, head_bias.reshape(1, V)
    )
    ids = input_ids.reshape(b).astype(jnp.int32)
    out = _sc_gather(m, ids, b, 1600)
    return out.reshape(bt, sl, V)


# speedup vs baseline: 4.6266x; 1.1016x over previous
import functools

import jax
import jax.numpy as jnp
from jax import lax
from jax.experimental import pallas as pl
from jax.experimental.pallas import tpu as pltpu
from jax.experimental.pallas import tpu_sc as plsc

V = 32
D = 8
NC = 2
NS = 16
NW = NC * NS
R_REP = 64


def _m_table_body(e_ref, ht_ref, b_ref, m_ref):
    m_ref[...] = (
        jnp.dot(e_ref[...], ht_ref[...], preferred_element_type=jnp.float32)
        + b_ref[...]
    )


def _compute_m_table(embed_weight, head_weight_t, head_bias_row):
    return pl.pallas_call(
        _m_table_body,
        grid=(R_REP,),
        in_specs=[
            pl.BlockSpec((V, D), lambda r: (0, 0)),
            pl.BlockSpec((D, V), lambda r: (0, 0)),
            pl.BlockSpec((1, V), lambda r: (0, 0)),
        ],
        out_specs=pl.BlockSpec((V, V), lambda r: (r, 0)),
        out_shape=jax.ShapeDtypeStruct((R_REP * V, V), jnp.float32),
    )(embed_weight, head_weight_t, head_bias_row)


@functools.partial(jax.jit, static_argnames=("b", "chunk"))
def _sc_gather(m, ids, b, chunk):
    b_per_w = b // NW
    n_chunks = b_per_w // chunk
    mesh = plsc.VectorSubcoreMesh(core_axis_name="c", subcore_axis_name="s")

    @functools.partial(
        pl.kernel,
        mesh=mesh,
        out_type=jax.ShapeDtypeStruct((b, V), jnp.float32),
        scratch_types=[
            pltpu.VMEM((chunk,), jnp.int32),
            pltpu.VMEM((chunk, V), jnp.float32),
            pltpu.SemaphoreType.DMA,
        ],
        compiler_params=pltpu.CompilerParams(use_tc_tiling_on_sc=False),
    )
    def gather_kernel(m_hbm, idx_hbm, out_hbm, idx_v, rows_v, sem):
        wid = lax.axis_index("s") * NC + lax.axis_index("c")
        base = wid * b_per_w
        rep_off = lax.iota(jnp.int32, 16) * V
        for c in range(n_chunks):
            off = base + c * chunk
            pltpu.sync_copy(idx_hbm.at[pl.ds(off, chunk)], idx_v)

            def spread(i, carry):
                sl = pl.ds(i * 16, 16)
                idx_v[sl] = idx_v[sl] + rep_off + (i % 4) * (16 * V)
                return carry

            lax.fori_loop(0, chunk // 16, spread, 0)
            pltpu.async_copy(m_hbm.at[idx_v], rows_v, sem).wait()
            pltpu.sync_copy(rows_v, out_hbm.at[pl.ds(off, chunk)])

    return gather_kernel(m, ids)


def kernel(input_ids, embed_weight, head_weight, head_bias):
    bt, sl = input_ids.shape
    b = bt * sl
    m = _compute_m_table(
        embed_weight, head_weight.T, head_bias.reshape(1, V)
    )
    ids = input_ids.reshape(b).astype(jnp.int32)
    out = _sc_gather(m, ids, b, 3200)
    return out.reshape(bt, sl, V)
